# baseline (speedup 1.0000x reference)
import functools

import jax
import jax.numpy as jnp
from jax import lax
from jax.experimental import pallas as pl
from jax.experimental.pallas import tpu as pltpu
from jax.experimental.pallas import tpu_sc as plsc

N_NODES = 10000
N_EDGES = 320000
HIDDEN = 128
HEADS = 8
HEAD_DIM = HIDDEN // HEADS
SCALE = HEAD_DIM ** (-0.5)

_ROW_BLK = 2000

_HHALF = HIDDEN // 2
_HH = HEADS // 2
_AW = 80
_C = 128
_NCHUNK = N_EDGES // _C
_NSUB = 16
_CPW = -(-_NCHUNK // _NSUB)
_LOOP = 2 * (-(-(_CPW + 2) // 2))


def _qkv_body(xs_ref, xd_ref, wq_ref, bq_ref, wk_ref, bk_ref, wv_ref, bv_ref,
              q_ref, k_ref, v_ref):
    xd = xd_ref[...]
    xs = xs_ref[...]
    dn = (((1,), (1,)), ((), ()))
    q = lax.dot_general(xd, wq_ref[...], dn,
                        preferred_element_type=jnp.float32) + bq_ref[...][None, :]
    k = lax.dot_general(xs, wk_ref[...], dn,
                        preferred_element_type=jnp.float32) + bk_ref[...][None, :]
    v = lax.dot_general(xs, wv_ref[...], dn,
                        preferred_element_type=jnp.float32) + bv_ref[...][None, :]
    q_ref[0] = q[:, :_HHALF]
    q_ref[1] = q[:, _HHALF:]
    k_ref[0] = k[:, :_HHALF]
    k_ref[1] = k[:, _HHALF:]
    v_ref[0] = v[:, :_HHALF]
    v_ref[1] = v[:, _HHALF:]


def _qkv(x_src, x_dst, Wq, bq, Wk, bk, Wv, bv):
    n = x_src.shape[0]
    grid = n // _ROW_BLK
    blk = pl.BlockSpec((_ROW_BLK, HIDDEN), lambda i: (i, 0))
    hblk = pl.BlockSpec((2, _ROW_BLK, _HHALF), lambda i: (0, i, 0))
    wspec = pl.BlockSpec((HIDDEN, HIDDEN), lambda i: (0, 0))
    bspec = pl.BlockSpec((HIDDEN,), lambda i: (0,))
    out = jax.ShapeDtypeStruct((2, n, _HHALF), jnp.float32)
    return pl.pallas_call(
        _qkv_body,
        grid=(grid,),
        in_specs=[blk, blk, wspec, bspec, wspec, bspec, wspec, bspec],
        out_specs=[hblk, hblk, hblk],
        out_shape=[out, out, out],
    )(x_src, x_dst, Wq, bq, Wk, bk, Wv, bv)


def _edge_sc(et, q2, k2, v2):
    n = q2.shape[1]
    za = jnp.zeros((n, _AW), jnp.float32)
    mesh = plsc.VectorSubcoreMesh(core_axis_name="c", subcore_axis_name="s",
                                  num_cores=2, num_subcores=_NSUB)

    @functools.partial(
        pl.kernel,
        out_type=jax.ShapeDtypeStruct((2, n, _AW), jnp.float32),
        mesh=mesh,
        compiler_params=pltpu.CompilerParams(needs_layout_passes=False,
                                             use_tc_tiling_on_sc=False,
                                             disable_bounds_checks=True),
        scratch_types=[
            pltpu.VMEM_SHARED((n, _AW), jnp.float32),
            [pltpu.VMEM((2, _C), jnp.int32)] * 2,
            [pltpu.VMEM((_C,), jnp.int32)] * 2,
            [pltpu.VMEM((_C, _HHALF), jnp.float32)] * 2,
            [pltpu.VMEM((_C, _HHALF), jnp.float32)] * 2,
            [pltpu.VMEM((_C, _HHALF), jnp.float32)] * 2,
            [pltpu.VMEM((_C, _AW), jnp.float32)] * 2,
            [pltpu.SemaphoreType.DMA] * 2,
            [pltpu.SemaphoreType.DMA] * 2,
            [pltpu.SemaphoreType.DMA] * 2,
            [pltpu.SemaphoreType.DMA] * 2,
            [pltpu.SemaphoreType.DMA] * 2,
        ],
    )
    def sc_kernel(et_h, q_h, k_h, v_h, za_h, acc_o,
                  acc_sh, ei, db, qr, kr, vr, wb, si, sq, sk, sv, ssc):
        cid = lax.axis_index("c")
        sid = lax.axis_index("s")

        @pl.when(sid == 0)
        def _zero():
            pltpu.sync_copy(za_h, acc_sh)

        iota = lax.iota(jnp.int32, 16)
        zero16 = jnp.zeros((16,), jnp.float32)

        for s in range(2):
            def zrow(r, c, s=s):
                wb[s][r, pl.ds(_HHALF, 16)] = zero16
                return c

            lax.fori_loop(0, _C, zrow, 0)

        plsc.subcore_barrier()

        qt = q_h.at[cid]
        kt = k_h.at[cid]
        vt = v_h.at[cid]
        base_g = sid * _CPW
        nv = jnp.minimum(_CPW, _NCHUNK - base_g)

        def compute_chunk(p):
            @plsc.parallel_loop(0, _C // 16, unroll=2)
            def m_body(m):
                rows = iota + m * 16
                for h in range(_HH):
                    acc = jnp.zeros((16,), jnp.float32)
                    for d in range(HEAD_DIM):
                        cols = jnp.full((16,), h * HEAD_DIM + d, jnp.int32)
                        acc += (plsc.load_gather(qr[p], [rows, cols])
                                * plsc.load_gather(kr[p], [rows, cols]))
                    e = jnp.exp(acc * SCALE)
                    plsc.store_scatter(
                        wb[p], [rows, jnp.full((16,), _HHALF + h, jnp.int32)], e)
                    for d in range(HEAD_DIM):
                        cols = jnp.full((16,), h * HEAD_DIM + d, jnp.int32)
                        vv = plsc.load_gather(vr[p], [rows, cols])
                        plsc.store_scatter(wb[p], [rows, cols], vv * e)

        pltpu.async_copy(et_h.at[base_g], ei[0], si[0])

        @pl.when(nv > 1)
        def _pre_idx1():
            pltpu.async_copy(et_h.at[base_g + 1], ei[1], si[1])

        pltpu.make_async_copy(et_h.at[base_g], ei[0], si[0]).wait()
        pltpu.async_copy(qt.at[ei[0].at[1]], qr[0], sq[0])
        pltpu.async_copy(kt.at[ei[0].at[0]], kr[0], sk[0])
        pltpu.async_copy(vt.at[ei[0].at[0]], vr[0], sv[0])

        def block_body(jb, carry):
            for d in range(2):
                j = jb * 2 + d
                e = d
                d1 = 1 - d
                e1 = 1 - e

                @pl.when((j >= 2) & (j - 2 < nv))
                def _drain(d=d):
                    pltpu.make_async_copy(wb[d], acc_sh.at[db[d]],
                                          ssc[d]).wait()

                @pl.when(j < nv)
                def _proc(d=d, e=e, j=j):
                    pltpu.make_async_copy(qt.at[ei[e].at[1]], qr[d], sq[d]).wait()
                    pltpu.make_async_copy(kt.at[ei[e].at[0]], kr[d], sk[d]).wait()
                    pltpu.make_async_copy(vt.at[ei[e].at[0]], vr[d], sv[d]).wait()
                    for m in range(_C // 16):
                        db[d][pl.ds(m * 16, 16)] = ei[e][1, pl.ds(m * 16, 16)]
                    compute_chunk(d)
                    pltpu.async_copy(wb[d], acc_sh.at[db[d]], ssc[d], add=True)

                @pl.when(j + 2 < nv)
                def _pidx(e=e, j=j):
                    pltpu.async_copy(et_h.at[base_g + j + 2], ei[e], si[e])

                @pl.when(j + 1 < nv)
                def _gath(d1=d1, e1=e1, j=j):
                    pltpu.make_async_copy(et_h.at[base_g + j + 1], ei[e1],
                                          si[e1]).wait()
                    pltpu.async_copy(qt.at[ei[e1].at[1]], qr[d1], sq[d1])
                    pltpu.async_copy(kt.at[ei[e1].at[0]], kr[d1], sk[d1])
                    pltpu.async_copy(vt.at[ei[e1].at[0]], vr[d1], sv[d1])

            return carry

        lax.fori_loop(0, _LOOP // 2, block_body, 0)

        plsc.subcore_barrier()

        start = sid * 624
        pltpu.sync_copy(acc_sh.at[pl.ds(start, 640)],
                        acc_o.at[cid].at[pl.ds(start, 640)])

    return sc_kernel(et, q2, k2, v2, za)


def _finish_body(acc_ref, sel_ref, wo_ref, bo_ref, out_ref):
    wv = jnp.concatenate([acc_ref[0, :, :_HHALF], acc_ref[1, :, :_HHALF]],
                         axis=1)
    sums = jnp.concatenate([acc_ref[0, :, _HHALF:_HHALF + _HH],
                            acc_ref[1, :, _HHALF:_HHALF + _HH]], axis=1)
    den = lax.dot_general(sums + 1e-8, sel_ref[...],
                          (((1,), (0,)), ((), ())),
                          preferred_element_type=jnp.float32)
    normed = wv / den
    dn = (((1,), (1,)), ((), ()))
    out_ref[...] = lax.dot_general(normed, wo_ref[...], dn,
                                   preferred_element_type=jnp.float32) + bo_ref[...][None, :]


def _finish(acc, Wo, bo):
    n = acc.shape[1]
    grid = n // _ROW_BLK
    sel = jnp.repeat(jnp.eye(HEADS, dtype=jnp.float32), HEAD_DIM, axis=1)
    return pl.pallas_call(
        _finish_body,
        grid=(grid,),
        in_specs=[
            pl.BlockSpec((2, _ROW_BLK, _AW), lambda i: (0, i, 0)),
            pl.BlockSpec((HEADS, HIDDEN), lambda i: (0, 0)),
            pl.BlockSpec((HIDDEN, HIDDEN), lambda i: (0, 0)),
            pl.BlockSpec((HIDDEN,), lambda i: (0,)),
        ],
        out_specs=pl.BlockSpec((_ROW_BLK, HIDDEN), lambda i: (i, 0)),
        out_shape=jax.ShapeDtypeStruct((n, HIDDEN), jnp.float32),
    )(acc, sel, Wo, bo)


def kernel(x_src, x_dst, edge_index, Wq, bq, Wk, bk, Wv, bv, Wo, bo):
    q2, k2, v2 = _qkv(x_src, x_dst, Wq, bq, Wk, bk, Wv, bv)
    et = edge_index.reshape(2, _NCHUNK, _C).transpose(1, 0, 2)
    pad = _NSUB * _CPW - _NCHUNK
    et = jnp.concatenate(
        [et, jnp.zeros((pad, 2, _C), jnp.int32)], axis=0)
    acc = _edge_sc(et, q2, k2, v2)
    return _finish(acc, Wo, bo)


# speedup vs baseline: 8.5764x; 8.5764x over previous
import functools

import jax
import jax.numpy as jnp
from jax import lax
from jax.experimental import pallas as pl
from jax.experimental.pallas import tpu as pltpu
from jax.experimental.pallas import tpu_sc as plsc

N_NODES = 10000
N_EDGES = 320000
HIDDEN = 128
HEADS = 8
HEAD_DIM = HIDDEN // HEADS
SCALE = HEAD_DIM ** (-0.5)

_ROW_BLK = 2000

_HHALF = HIDDEN // 2
_HH = HEADS // 2
_AW = 80
_C = 128
_NCHUNK = N_EDGES // _C
_NSUB = 16
_CPW = -(-_NCHUNK // _NSUB)
_LOOP = 2 * (-(-(_CPW + 2) // 2))


def _qkv_body(xs_ref, xd_ref, wq_ref, bq_ref, wk_ref, bk_ref, wv_ref, bv_ref,
              q_ref, k_ref, v_ref):
    xd = xd_ref[...]
    xs = xs_ref[...]
    dn = (((1,), (1,)), ((), ()))
    q = lax.dot_general(xd, wq_ref[...], dn,
                        preferred_element_type=jnp.float32) + bq_ref[...][None, :]
    k = lax.dot_general(xs, wk_ref[...], dn,
                        preferred_element_type=jnp.float32) + bk_ref[...][None, :]
    v = lax.dot_general(xs, wv_ref[...], dn,
                        preferred_element_type=jnp.float32) + bv_ref[...][None, :]
    q_ref[0] = q[:, :_HHALF].astype(jnp.bfloat16)
    q_ref[1] = q[:, _HHALF:].astype(jnp.bfloat16)
    k_ref[0] = k[:, :_HHALF].astype(jnp.bfloat16)
    k_ref[1] = k[:, _HHALF:].astype(jnp.bfloat16)
    v_ref[0] = v[:, :_HHALF]
    v_ref[1] = v[:, _HHALF:]


def _qkv(x_src, x_dst, Wq, bq, Wk, bk, Wv, bv):
    n = x_src.shape[0]
    grid = n // _ROW_BLK
    blk = pl.BlockSpec((_ROW_BLK, HIDDEN), lambda i: (i, 0))
    hblk = pl.BlockSpec((2, _ROW_BLK, _HHALF), lambda i: (0, i, 0))
    wspec = pl.BlockSpec((HIDDEN, HIDDEN), lambda i: (0, 0))
    bspec = pl.BlockSpec((HIDDEN,), lambda i: (0,))
    out16 = jax.ShapeDtypeStruct((2, n, _HHALF), jnp.bfloat16)
    out32 = jax.ShapeDtypeStruct((2, n, _HHALF), jnp.float32)
    return pl.pallas_call(
        _qkv_body,
        grid=(grid,),
        in_specs=[blk, blk, wspec, bspec, wspec, bspec, wspec, bspec],
        out_specs=[hblk, hblk, hblk],
        out_shape=[out16, out16, out32],
    )(x_src, x_dst, Wq, bq, Wk, bk, Wv, bv)


def _edge_sc(et, q2, k2, v2):
    n = q2.shape[1]
    za = jnp.zeros((n, _AW), jnp.float32)
    mesh = plsc.VectorSubcoreMesh(core_axis_name="c", subcore_axis_name="s",
                                  num_cores=2, num_subcores=_NSUB)

    @functools.partial(
        pl.kernel,
        out_type=jax.ShapeDtypeStruct((2, n, _AW), jnp.float32),
        mesh=mesh,
        compiler_params=pltpu.CompilerParams(needs_layout_passes=False,
                                             use_tc_tiling_on_sc=False,
                                             disable_bounds_checks=True),
        scratch_types=[
            pltpu.VMEM_SHARED((n, _AW), jnp.float32),
            [pltpu.VMEM((2, _C), jnp.int32)] * 2,
            [pltpu.VMEM((_C,), jnp.int32)] * 2,
            [pltpu.VMEM((_C, _HHALF), jnp.bfloat16)] * 2,
            [pltpu.VMEM((_C, _HHALF), jnp.bfloat16)] * 2,
            [pltpu.VMEM((_C, _HHALF), jnp.float32)] * 2,
            [pltpu.VMEM((_C, _AW), jnp.float32)] * 2,
            [pltpu.SemaphoreType.DMA] * 2,
            [pltpu.SemaphoreType.DMA] * 2,
            [pltpu.SemaphoreType.DMA] * 2,
            [pltpu.SemaphoreType.DMA] * 2,
            [pltpu.SemaphoreType.DMA] * 2,
        ],
    )
    def sc_kernel(et_h, q_h, k_h, v_h, za_h, acc_o,
                  acc_sh, ei, db, qr, kr, vr, wb, si, sq, sk, sv, ssc):
        cid = lax.axis_index("c")
        sid = lax.axis_index("s")

        @pl.when(sid == 0)
        def _zero():
            pltpu.sync_copy(za_h, acc_sh)

        iota = lax.iota(jnp.int32, 16)

        plsc.subcore_barrier()

        qt = q_h.at[cid]
        kt = k_h.at[cid]
        vt = v_h.at[cid]
        base_g = sid * _CPW
        nv = jnp.minimum(_CPW, _NCHUNK - base_g)

        idx15 = jnp.full((16,), 15, jnp.int32)

        def compute_chunk(p):
            @plsc.parallel_loop(0, _C, unroll=2)
            def e_body(er):
                sumrow = jnp.zeros((16,), jnp.float32)
                for hh in range(_HH // 2):
                    slp = pl.ds(hh * 32, 32)
                    qa, qb = plsc.unpack(qr[p][er, slp],
                                         format=plsc.PackFormat.INTERLEAVED,
                                         preferred_element_type=jnp.float32)
                    ka, kb = plsc.unpack(kr[p][er, slp],
                                         format=plsc.PackFormat.INTERLEAVED,
                                         preferred_element_type=jnp.float32)
                    for h, qx, kx in ((2 * hh, qa, ka), (2 * hh + 1, qb, kb)):
                        sl = pl.ds(h * HEAD_DIM, 16)
                        tot = jnp.sum(qx * kx)
                        e_b = jnp.exp(jnp.full((16,), tot * SCALE, jnp.float32))
                        wb[p][er, sl] = vr[p][er, sl] * e_b
                        sumrow = jnp.where(iota == h, e_b, sumrow)
                wb[p][er, pl.ds(_HHALF, 16)] = sumrow

        pltpu.async_copy(et_h.at[base_g], ei[0], si[0])

        @pl.when(nv > 1)
        def _pre_idx1():
            pltpu.async_copy(et_h.at[base_g + 1], ei[1], si[1])

        pltpu.make_async_copy(et_h.at[base_g], ei[0], si[0]).wait()
        pltpu.async_copy(qt.at[ei[0].at[1]], qr[0], sq[0])
        pltpu.async_copy(kt.at[ei[0].at[0]], kr[0], sk[0])
        pltpu.async_copy(vt.at[ei[0].at[0]], vr[0], sv[0])

        def block_body(jb, carry):
            for d in range(2):
                j = jb * 2 + d
                e = d
                d1 = 1 - d
                e1 = 1 - e

                @pl.when((j >= 2) & (j - 2 < nv))
                def _drain(d=d):
                    pltpu.make_async_copy(wb[d], acc_sh.at[db[d]],
                                          ssc[d]).wait()

                @pl.when(j + 1 < nv)
                def _gath(d1=d1, e1=e1, j=j):
                    pltpu.make_async_copy(et_h.at[base_g + j + 1], ei[e1],
                                          si[e1]).wait()
                    pltpu.async_copy(qt.at[ei[e1].at[1]], qr[d1], sq[d1])
                    pltpu.async_copy(kt.at[ei[e1].at[0]], kr[d1], sk[d1])
                    pltpu.async_copy(vt.at[ei[e1].at[0]], vr[d1], sv[d1])

                @pl.when(j < nv)
                def _proc(d=d, e=e, j=j):
                    pltpu.make_async_copy(qt.at[ei[e].at[1]], qr[d], sq[d]).wait()
                    pltpu.make_async_copy(kt.at[ei[e].at[0]], kr[d], sk[d]).wait()
                    pltpu.make_async_copy(vt.at[ei[e].at[0]], vr[d], sv[d]).wait()
                    for m in range(_C // 16):
                        db[d][pl.ds(m * 16, 16)] = ei[e][1, pl.ds(m * 16, 16)]

                    @pl.when(j + 2 < nv)
                    def _pidx(e=e, j=j):
                        pltpu.async_copy(et_h.at[base_g + j + 2], ei[e], si[e])

                    compute_chunk(d)
                    pltpu.async_copy(wb[d], acc_sh.at[db[d]], ssc[d], add=True)

            return carry

        lax.fori_loop(0, _LOOP // 2, block_body, 0)

        plsc.subcore_barrier()

        start = sid * 624
        pltpu.sync_copy(acc_sh.at[pl.ds(start, 640)],
                        acc_o.at[cid].at[pl.ds(start, 640)])

    return sc_kernel(et, q2, k2, v2, za)


def _finish_body(acc_ref, sel_ref, wo_ref, bo_ref, out_ref):
    wv = jnp.concatenate([acc_ref[0, :, :_HHALF], acc_ref[1, :, :_HHALF]],
                         axis=1)
    sums = jnp.concatenate([acc_ref[0, :, _HHALF:_HHALF + _HH],
                            acc_ref[1, :, _HHALF:_HHALF + _HH]], axis=1)
    den = lax.dot_general(sums + 1e-8, sel_ref[...],
                          (((1,), (0,)), ((), ())),
                          preferred_element_type=jnp.float32)
    normed = wv / den
    dn = (((1,), (1,)), ((), ()))
    out_ref[...] = lax.dot_general(normed, wo_ref[...], dn,
                                   preferred_element_type=jnp.float32) + bo_ref[...][None, :]


def _finish(acc, Wo, bo):
    n = acc.shape[1]
    grid = n // _ROW_BLK
    sel = jnp.repeat(jnp.eye(HEADS, dtype=jnp.float32), HEAD_DIM, axis=1)
    return pl.pallas_call(
        _finish_body,
        grid=(grid,),
        in_specs=[
            pl.BlockSpec((2, _ROW_BLK, _AW), lambda i: (0, i, 0)),
            pl.BlockSpec((HEADS, HIDDEN), lambda i: (0, 0)),
            pl.BlockSpec((HIDDEN, HIDDEN), lambda i: (0, 0)),
            pl.BlockSpec((HIDDEN,), lambda i: (0,)),
        ],
        out_specs=pl.BlockSpec((_ROW_BLK, HIDDEN), lambda i: (i, 0)),
        out_shape=jax.ShapeDtypeStruct((n, HIDDEN), jnp.float32),
    )(acc, sel, Wo, bo)


def _interleave_perm():
    perm = [0] * HIDDEN
    for c in range(2):
        for hh in range(2):
            for dd in range(HEAD_DIM):
                perm[c * 64 + hh * 32 + 2 * dd] = (4 * c + 2 * hh) * 16 + dd
                perm[c * 64 + hh * 32 + 2 * dd + 1] = (4 * c + 2 * hh + 1) * 16 + dd
    return jnp.asarray(perm, jnp.int32)


def kernel(x_src, x_dst, edge_index, Wq, bq, Wk, bk, Wv, bv, Wo, bo):
    perm = _interleave_perm()
    Wq = Wq[perm]
    bq = bq[perm]
    Wk = Wk[perm]
    bk = bk[perm]
    q2, k2, v2 = _qkv(x_src, x_dst, Wq, bq, Wk, bk, Wv, bv)
    et = edge_index.reshape(2, _NCHUNK, _C).transpose(1, 0, 2)
    pad = _NSUB * _CPW - _NCHUNK
    et = jnp.concatenate(
        [et, jnp.zeros((pad, 2, _C), jnp.int32)], axis=0)
    acc = _edge_sc(et, q2, k2, v2)
    return _finish(acc, Wo, bo)
